# baseline (speedup 1.0000x reference)
import functools

import jax
import jax.numpy as jnp
from jax import lax
from jax.experimental import pallas as pl
from jax.experimental.pallas import tpu as pltpu
from jax.experimental.pallas import tpu_sc as plsc

NC = 2
NS = 16
NW = NC * NS
CHUNK = 128


def _round_up(a, b):
    return (a + b - 1) // b * b


def _matmul_z(x, w):
    v_tot, inc = x.shape
    k_tot, _, outc = w.shape
    vb = 400 if v_tot % 400 == 0 else v_tot
    nv = v_tot // vb

    def body(x_ref, w_ref, o_ref):
        xb = x_ref[...].astype(jnp.bfloat16)
        for k in range(k_tot):
            o_ref[k] = jnp.dot(xb, w_ref[k].astype(jnp.bfloat16),
                               preferred_element_type=jnp.float32)

    return pl.pallas_call(
        body,
        grid=(nv,),
        in_specs=[
            pl.BlockSpec((vb, inc), lambda v: (v, 0)),
            pl.BlockSpec((k_tot, inc, outc), lambda v: (0, 0, 0)),
        ],
        out_specs=pl.BlockSpec((k_tot, vb, outc), lambda v: (0, v, 0)),
        out_shape=jax.ShapeDtypeStruct((k_tot, v_tot, outc), jnp.float32),
    )(x, w)


def _add_partials4(p0, p1, p2, p3):
    v_tot, outc = p0.shape
    vb = 2000 if v_tot % 2000 == 0 else v_tot
    nv = v_tot // vb

    def body(a_ref, b_ref, c_ref, d_ref, o_ref):
        o_ref[...] = ((a_ref[...] + b_ref[...])
                      + (c_ref[...] + d_ref[...]))

    spec = pl.BlockSpec((vb, outc), lambda v: (v, 0))
    return pl.pallas_call(
        body,
        grid=(nv,),
        in_specs=[spec, spec, spec, spec],
        out_specs=spec,
        out_shape=jax.ShapeDtypeStruct((v_tot, outc), jnp.float32),
    )(p0, p1, p2, p3)


def _make_sc_scatter(outc, acc_rows, n_l):
    rows_per_sub = acc_rows // NS
    mesh = plsc.VectorSubcoreMesh(core_axis_name="c", subcore_axis_name="s")

    @functools.partial(
        pl.kernel,
        mesh=mesh,
        out_type=jax.ShapeDtypeStruct((NC, acc_rows, outc), jnp.float32),
        scratch_types=[
            pltpu.VMEM((n_l, CHUNK), jnp.int32),
            pltpu.VMEM((n_l, CHUNK), jnp.int32),
            pltpu.VMEM((CHUNK, outc), jnp.float32),
            pltpu.VMEM((CHUNK, outc), jnp.float32),
            pltpu.VMEM_SHARED((acc_rows, outc), jnp.float32),
            pltpu.SemaphoreType.DMA,
            pltpu.SemaphoreType.DMA,
        ],
    )
    def sc_scatter(z_hbm, gidx_hbm, sidx_hbm, out_hbm,
                   gidx_v, sidx_v, rows_a, rows_b, acc, sem_a, sem_b):
        c = lax.axis_index("c")
        s = lax.axis_index("s")
        w = c * NS + s
        zero16 = jnp.zeros((16,), jnp.float32)

        def zrow(i, carry):
            for l in range(outc // 16):
                rows_a[i, pl.ds(l * 16, 16)] = zero16
            return carry

        lax.fori_loop(0, CHUNK, zrow, 0)
        for off in range(0, rows_per_sub, CHUNK):
            sz = min(CHUNK, rows_per_sub - off)
            pltpu.sync_copy(
                rows_a.at[pl.ds(0, sz)],
                acc.at[pl.ds(s * rows_per_sub + off, sz)])
        pltpu.sync_copy(gidx_hbm.at[w], gidx_v)
        pltpu.sync_copy(sidx_hbm.at[w], sidx_v)
        plsc.subcore_barrier()

        pltpu.async_copy(z_hbm.at[gidx_v.at[0]], rows_a, sem_a)

        def body(i, carry):
            j = 2 * i
            pltpu.async_copy(z_hbm.at[gidx_v.at[j + 1]], rows_b, sem_b)
            pltpu.make_async_copy(z_hbm.at[gidx_v.at[j]], rows_a,
                                  sem_a).wait()
            pltpu.sync_copy(rows_a, acc.at[sidx_v.at[j]], add=True)

            @pl.when(j + 2 < n_l)
            def _():
                pltpu.async_copy(z_hbm.at[gidx_v.at[j + 2]], rows_a, sem_a)

            pltpu.make_async_copy(z_hbm.at[gidx_v.at[j + 1]], rows_b,
                                  sem_b).wait()
            pltpu.sync_copy(rows_b, acc.at[sidx_v.at[j + 1]], add=True)
            return carry

        lax.fori_loop(0, n_l // 2, body, 0)
        plsc.subcore_barrier()
        pltpu.sync_copy(acc.at[pl.ds(s * rows_per_sub, rows_per_sub)],
                        out_hbm.at[c, pl.ds(s * rows_per_sub,
                                            rows_per_sub)])

    return sc_scatter


def _make_idx(nbr_in_g, nbr_out_g, n_vox, acc_rows):
    kg, e_tot = nbr_in_g.shape
    n_pairs = kg * e_tot
    pairs_pad = _round_up(n_pairs, NW * CHUNK * 2)
    n_l = pairs_pad // (NW * CHUNK)
    koff = (jnp.arange(kg, dtype=jnp.int32) * n_vox)[:, None]
    gflat = (nbr_in_g.astype(jnp.int32) + koff).reshape(-1)
    gidx = jnp.concatenate(
        [gflat, jnp.zeros((pairs_pad - n_pairs,), jnp.int32)]
    ).reshape(NW, n_l, CHUNK)
    n_dump = acc_rows - n_vox
    pad_dst = n_vox + jnp.arange(pairs_pad - n_pairs, dtype=jnp.int32) % n_dump
    sidx = jnp.concatenate(
        [nbr_out_g.astype(jnp.int32).reshape(-1), pad_dst]
    ).reshape(NW, n_l, CHUNK)
    return gidx, sidx, n_l


def kernel(x, nbr_in, nbr_out, kernel):
    n_vox, inc = x.shape
    k_tot, e_tot = nbr_in.shape
    outc = kernel.shape[-1]
    acc_rows = _round_up(n_vox + 1, NS * 8)
    h = k_tot // 2

    za = _matmul_z(x, kernel[:h]).reshape(h * n_vox, outc)
    zb = _matmul_z(x, kernel[h:]).reshape((k_tot - h) * n_vox, outc)

    gidx_a, sidx_a, n_la = _make_idx(nbr_in[:h], nbr_out[:h],
                                     n_vox, acc_rows)
    gidx_b, sidx_b, n_lb = _make_idx(nbr_in[h:], nbr_out[h:],
                                     n_vox, acc_rows)

    pa = _make_sc_scatter(outc, acc_rows, n_la)(za, gidx_a, sidx_a)
    pb = _make_sc_scatter(outc, acc_rows, n_lb)(zb, gidx_b, sidx_b)

    return _add_partials4(pa[0, :n_vox], pa[1, :n_vox],
                          pb[0, :n_vox], pb[1, :n_vox])


# speedup vs baseline: 4.3392x; 4.3392x over previous
import functools

import jax
import jax.numpy as jnp
from jax import lax
from jax.experimental import pallas as pl
from jax.experimental.pallas import tpu as pltpu
from jax.experimental.pallas import tpu_sc as plsc

NC = 2
NS = 16
NW = NC * NS
CHUNK = 128


def _round_up(a, b):
    return (a + b - 1) // b * b


def _matmul_z(x, w):
    v_tot, inc = x.shape
    k_tot, _, outc = w.shape
    vb = 400 if v_tot % 400 == 0 else v_tot
    nv = v_tot // vb

    def body(x_ref, w_ref, o_ref):
        xb = x_ref[...].astype(jnp.bfloat16)
        for k in range(k_tot):
            o_ref[k] = jnp.dot(xb, w_ref[k].astype(jnp.bfloat16),
                               preferred_element_type=jnp.float32)

    return pl.pallas_call(
        body,
        grid=(nv,),
        in_specs=[
            pl.BlockSpec((vb, inc), lambda v: (v, 0)),
            pl.BlockSpec((k_tot, inc, outc), lambda v: (0, 0, 0)),
        ],
        out_specs=pl.BlockSpec((k_tot, vb, outc), lambda v: (0, v, 0)),
        out_shape=jax.ShapeDtypeStruct((k_tot, v_tot, outc), jnp.float32),
    )(x, w)


def _add_partials4(p0, p1, p2, p3):
    v_tot, outc = p0.shape
    vb = 2000 if v_tot % 2000 == 0 else v_tot
    nv = v_tot // vb

    def body(a_ref, b_ref, c_ref, d_ref, o_ref):
        o_ref[...] = ((a_ref[...] + b_ref[...])
                      + (c_ref[...] + d_ref[...]))

    spec = pl.BlockSpec((vb, outc), lambda v: (v, 0))
    return pl.pallas_call(
        body,
        grid=(nv,),
        in_specs=[spec, spec, spec, spec],
        out_specs=spec,
        out_shape=jax.ShapeDtypeStruct((v_tot, outc), jnp.float32),
    )(p0, p1, p2, p3)


def _make_sc_scatter(outc, acc_rows, n_l):
    rows_per_sub = acc_rows // NS
    mesh = plsc.VectorSubcoreMesh(core_axis_name="c", subcore_axis_name="s")

    @functools.partial(
        pl.kernel,
        mesh=mesh,
        out_type=jax.ShapeDtypeStruct((NC, acc_rows, outc), jnp.float32),
        scratch_types=[
            pltpu.VMEM((n_l, CHUNK), jnp.int32),
            pltpu.VMEM((n_l, CHUNK), jnp.int32),
            pltpu.VMEM((CHUNK, outc), jnp.float32),
            pltpu.VMEM((CHUNK, outc), jnp.float32),
            pltpu.VMEM_SHARED((acc_rows, outc), jnp.float32),
            pltpu.SemaphoreType.DMA,
            pltpu.SemaphoreType.DMA,
        ],
    )
    def sc_scatter(z_hbm, gidx_hbm, sidx_hbm, out_hbm,
                   gidx_v, sidx_v, rows_a, rows_b, acc, sem_a, sem_b):
        c = lax.axis_index("c")
        s = lax.axis_index("s")
        w = c * NS + s
        zero16 = jnp.zeros((16,), jnp.float32)

        def zrow(i, carry):
            for l in range(outc // 16):
                rows_a[i, pl.ds(l * 16, 16)] = zero16
            return carry

        lax.fori_loop(0, CHUNK, zrow, 0)
        for off in range(0, rows_per_sub, CHUNK):
            sz = min(CHUNK, rows_per_sub - off)
            pltpu.sync_copy(
                rows_a.at[pl.ds(0, sz)],
                acc.at[pl.ds(s * rows_per_sub + off, sz)])
        pltpu.sync_copy(gidx_hbm.at[w], gidx_v)
        pltpu.sync_copy(sidx_hbm.at[w], sidx_v)
        plsc.subcore_barrier()

        pltpu.async_copy(z_hbm.at[gidx_v.at[0]], rows_a, sem_a)

        def body(i, carry):
            j = 2 * i
            pltpu.async_copy(z_hbm.at[gidx_v.at[j + 1]], rows_b, sem_b)
            pltpu.make_async_copy(z_hbm.at[gidx_v.at[j]], rows_a,
                                  sem_a).wait()
            pltpu.sync_copy(rows_a, acc.at[sidx_v.at[j]], add=True)

            @pl.when(j + 2 < n_l)
            def _():
                pltpu.async_copy(z_hbm.at[gidx_v.at[j + 2]], rows_a, sem_a)

            pltpu.make_async_copy(z_hbm.at[gidx_v.at[j + 1]], rows_b,
                                  sem_b).wait()
            pltpu.sync_copy(rows_b, acc.at[sidx_v.at[j + 1]], add=True)
            return carry

        lax.fori_loop(0, n_l // 2, body, 0)
        plsc.subcore_barrier()
        pltpu.sync_copy(acc.at[pl.ds(s * rows_per_sub, rows_per_sub)],
                        out_hbm.at[c, pl.ds(s * rows_per_sub,
                                            rows_per_sub)])

    return sc_scatter


def _make_idx(nbr_in_g, nbr_out_g, n_vox, acc_rows):
    kg, e_tot = nbr_in_g.shape
    n_pairs = kg * e_tot
    pairs_pad = _round_up(n_pairs, NW * CHUNK * 2)
    n_l = pairs_pad // (NW * CHUNK)
    koff = (jnp.arange(kg, dtype=jnp.int32) * n_vox)[:, None]
    gflat = (nbr_in_g.astype(jnp.int32) + koff).reshape(-1)
    pad_src = jnp.arange(pairs_pad - n_pairs, dtype=jnp.int32) % (kg * n_vox)
    gidx = jnp.concatenate([gflat, pad_src]).reshape(NW, n_l, CHUNK)
    n_dump = acc_rows - n_vox
    pad_dst = n_vox + jnp.arange(pairs_pad - n_pairs, dtype=jnp.int32) % n_dump
    sidx = jnp.concatenate(
        [nbr_out_g.astype(jnp.int32).reshape(-1), pad_dst]
    ).reshape(NW, n_l, CHUNK)
    return gidx, sidx, n_l


def kernel(x, nbr_in, nbr_out, kernel):
    n_vox, inc = x.shape
    k_tot, e_tot = nbr_in.shape
    outc = kernel.shape[-1]
    acc_rows = _round_up(n_vox + 1, NS * 8)
    h = k_tot // 2

    za = _matmul_z(x, kernel[:h]).reshape(h * n_vox, outc)
    zb = _matmul_z(x, kernel[h:]).reshape((k_tot - h) * n_vox, outc)

    gidx_a, sidx_a, n_la = _make_idx(nbr_in[:h], nbr_out[:h],
                                     n_vox, acc_rows)
    gidx_b, sidx_b, n_lb = _make_idx(nbr_in[h:], nbr_out[h:],
                                     n_vox, acc_rows)

    pa = _make_sc_scatter(outc, acc_rows, n_la)(za, gidx_a, sidx_a)
    pb = _make_sc_scatter(outc, acc_rows, n_lb)(zb, gidx_b, sidx_b)

    return _add_partials4(pa[0, :n_vox], pa[1, :n_vox],
                          pb[0, :n_vox], pb[1, :n_vox])


# speedup vs baseline: 4.6226x; 1.0653x over previous
import functools

import jax
import jax.numpy as jnp
from jax import lax
from jax.experimental import pallas as pl
from jax.experimental.pallas import tpu as pltpu
from jax.experimental.pallas import tpu_sc as plsc

NC = 2
NS = 16
NW = NC * NS
CHUNK = 128


def _round_up(a, b):
    return (a + b - 1) // b * b


def _matmul_z(x, w):
    v_tot, inc = x.shape
    k_tot, _, outc = w.shape
    vb = 400 if v_tot % 400 == 0 else v_tot
    nv = v_tot // vb

    def body(x_ref, w_ref, o_ref):
        xb = x_ref[...].astype(jnp.bfloat16)
        for k in range(k_tot):
            o_ref[k] = jnp.dot(xb, w_ref[k].astype(jnp.bfloat16),
                               preferred_element_type=jnp.float32)

    return pl.pallas_call(
        body,
        grid=(nv,),
        in_specs=[
            pl.BlockSpec((vb, inc), lambda v: (v, 0)),
            pl.BlockSpec((k_tot, inc, outc), lambda v: (0, 0, 0)),
        ],
        out_specs=pl.BlockSpec((k_tot, vb, outc), lambda v: (0, v, 0)),
        out_shape=jax.ShapeDtypeStruct((k_tot, v_tot, outc), jnp.float32),
    )(x, w)


def _add_partials4(pa, pb, n_vox):
    _, _, outc = pa.shape
    vb = 2000 if n_vox % 2000 == 0 else n_vox
    nv = n_vox // vb

    def body(a_ref, b_ref, o_ref):
        o_ref[...] = ((a_ref[0] + a_ref[1])
                      + (b_ref[0] + b_ref[1]))

    spec = pl.BlockSpec((NC, vb, outc), lambda v: (0, v, 0))
    return pl.pallas_call(
        body,
        grid=(nv,),
        in_specs=[spec, spec],
        out_specs=pl.BlockSpec((vb, outc), lambda v: (v, 0)),
        out_shape=jax.ShapeDtypeStruct((n_vox, outc), jnp.float32),
    )(pa, pb)


def _make_sc_scatter(outc, acc_rows, n_l):
    rows_per_sub = acc_rows // NS
    mesh = plsc.VectorSubcoreMesh(core_axis_name="c", subcore_axis_name="s")

    @functools.partial(
        pl.kernel,
        mesh=mesh,
        out_type=jax.ShapeDtypeStruct((NC, acc_rows, outc), jnp.float32),
        scratch_types=[
            pltpu.VMEM((n_l, CHUNK), jnp.int32),
            pltpu.VMEM((n_l, CHUNK), jnp.int32),
            pltpu.VMEM((CHUNK, outc), jnp.float32),
            pltpu.VMEM((CHUNK, outc), jnp.float32),
            pltpu.VMEM_SHARED((acc_rows, outc), jnp.float32),
            pltpu.SemaphoreType.DMA,
            pltpu.SemaphoreType.DMA,
        ],
    )
    def sc_scatter(z_hbm, gidx_hbm, sidx_hbm, out_hbm,
                   gidx_v, sidx_v, rows_a, rows_b, acc, sem_a, sem_b):
        c = lax.axis_index("c")
        s = lax.axis_index("s")
        w = c * NS + s
        zero16 = jnp.zeros((16,), jnp.float32)

        def zrow(i, carry):
            for l in range(outc // 16):
                rows_a[i, pl.ds(l * 16, 16)] = zero16
            return carry

        lax.fori_loop(0, CHUNK, zrow, 0)
        for off in range(0, rows_per_sub, CHUNK):
            sz = min(CHUNK, rows_per_sub - off)
            pltpu.sync_copy(
                rows_a.at[pl.ds(0, sz)],
                acc.at[pl.ds(s * rows_per_sub + off, sz)])
        pltpu.sync_copy(gidx_hbm.at[w], gidx_v)
        pltpu.sync_copy(sidx_hbm.at[w], sidx_v)
        plsc.subcore_barrier()

        pltpu.async_copy(z_hbm.at[gidx_v.at[0]], rows_a, sem_a)

        def body(i, carry):
            j = 2 * i
            pltpu.async_copy(z_hbm.at[gidx_v.at[j + 1]], rows_b, sem_b)
            pltpu.make_async_copy(z_hbm.at[gidx_v.at[j]], rows_a,
                                  sem_a).wait()
            pltpu.sync_copy(rows_a, acc.at[sidx_v.at[j]], add=True)

            @pl.when(j + 2 < n_l)
            def _():
                pltpu.async_copy(z_hbm.at[gidx_v.at[j + 2]], rows_a, sem_a)

            pltpu.make_async_copy(z_hbm.at[gidx_v.at[j + 1]], rows_b,
                                  sem_b).wait()
            pltpu.sync_copy(rows_b, acc.at[sidx_v.at[j + 1]], add=True)
            return carry

        lax.fori_loop(0, n_l // 2, body, 0)
        plsc.subcore_barrier()
        pltpu.sync_copy(acc.at[pl.ds(s * rows_per_sub, rows_per_sub)],
                        out_hbm.at[c, pl.ds(s * rows_per_sub,
                                            rows_per_sub)])

    return sc_scatter


def _make_idx(nbr_in_g, nbr_out_g, n_vox, acc_rows):
    kg, e_tot = nbr_in_g.shape
    n_pairs = kg * e_tot
    pairs_pad = _round_up(n_pairs, NW * CHUNK * 2)
    n_l = pairs_pad // (NW * CHUNK)
    koff = (jnp.arange(kg, dtype=jnp.int32) * n_vox)[:, None]
    gflat = (nbr_in_g.astype(jnp.int32) + koff).reshape(-1)
    pad_src = jnp.arange(pairs_pad - n_pairs, dtype=jnp.int32) % (kg * n_vox)
    gidx = jnp.concatenate([gflat, pad_src]).reshape(NW, n_l, CHUNK)
    n_dump = acc_rows - n_vox
    pad_dst = n_vox + jnp.arange(pairs_pad - n_pairs, dtype=jnp.int32) % n_dump
    sidx = jnp.concatenate(
        [nbr_out_g.astype(jnp.int32).reshape(-1), pad_dst]
    ).reshape(NW, n_l, CHUNK)
    return gidx, sidx, n_l


def kernel(x, nbr_in, nbr_out, kernel):
    n_vox, inc = x.shape
    k_tot, e_tot = nbr_in.shape
    outc = kernel.shape[-1]
    acc_rows = _round_up(n_vox + 1, NS * 8)
    h = k_tot // 2

    za = _matmul_z(x, kernel[:h]).reshape(h * n_vox, outc)
    zb = _matmul_z(x, kernel[h:]).reshape((k_tot - h) * n_vox, outc)

    gidx_a, sidx_a, n_la = _make_idx(nbr_in[:h], nbr_out[:h],
                                     n_vox, acc_rows)
    gidx_b, sidx_b, n_lb = _make_idx(nbr_in[h:], nbr_out[h:],
                                     n_vox, acc_rows)

    pa = _make_sc_scatter(outc, acc_rows, n_la)(za, gidx_a, sidx_a)
    pb = _make_sc_scatter(outc, acc_rows, n_lb)(zb, gidx_b, sidx_b)

    return _add_partials4(pa, pb, n_vox)


# speedup vs baseline: 4.6978x; 1.0163x over previous
import functools

import jax
import jax.numpy as jnp
from jax import lax
from jax.experimental import pallas as pl
from jax.experimental.pallas import tpu as pltpu
from jax.experimental.pallas import tpu_sc as plsc

NC = 2
NS = 16
NW = NC * NS
CHUNK = 128


def _round_up(a, b):
    return (a + b - 1) // b * b


def _matmul_z(x, w):
    v_tot, inc = x.shape
    k_tot, _, outc = w.shape
    vb = 1000 if v_tot % 1000 == 0 else v_tot
    nv = v_tot // vb

    def body(x_ref, w_ref, o_ref):
        xb = x_ref[...].astype(jnp.bfloat16)
        for k in range(k_tot):
            o_ref[k] = jnp.dot(xb, w_ref[k].astype(jnp.bfloat16),
                               preferred_element_type=jnp.float32)

    return pl.pallas_call(
        body,
        grid=(nv,),
        in_specs=[
            pl.BlockSpec((vb, inc), lambda v: (v, 0)),
            pl.BlockSpec((k_tot, inc, outc), lambda v: (0, 0, 0)),
        ],
        out_specs=pl.BlockSpec((k_tot, vb, outc), lambda v: (0, v, 0)),
        out_shape=jax.ShapeDtypeStruct((k_tot, v_tot, outc), jnp.float32),
    )(x, w)


def _add_partials4(pa, pb, n_vox):
    _, _, outc = pa.shape
    vb = 2000 if n_vox % 2000 == 0 else n_vox
    nv = n_vox // vb

    def body(a_ref, b_ref, o_ref):
        o_ref[...] = ((a_ref[0] + a_ref[1])
                      + (b_ref[0] + b_ref[1]))

    spec = pl.BlockSpec((NC, vb, outc), lambda v: (0, v, 0))
    return pl.pallas_call(
        body,
        grid=(nv,),
        in_specs=[spec, spec],
        out_specs=pl.BlockSpec((vb, outc), lambda v: (v, 0)),
        out_shape=jax.ShapeDtypeStruct((n_vox, outc), jnp.float32),
    )(pa, pb)


def _make_sc_scatter(outc, acc_rows, n_l):
    rows_per_sub = acc_rows // NS
    mesh = plsc.VectorSubcoreMesh(core_axis_name="c", subcore_axis_name="s")

    @functools.partial(
        pl.kernel,
        mesh=mesh,
        out_type=jax.ShapeDtypeStruct((NC, acc_rows, outc), jnp.float32),
        scratch_types=[
            pltpu.VMEM((n_l, CHUNK), jnp.int32),
            pltpu.VMEM((n_l, CHUNK), jnp.int32),
            pltpu.VMEM((CHUNK, outc), jnp.float32),
            pltpu.VMEM((CHUNK, outc), jnp.float32),
            pltpu.VMEM_SHARED((acc_rows, outc), jnp.float32),
            pltpu.SemaphoreType.DMA,
            pltpu.SemaphoreType.DMA,
        ],
    )
    def sc_scatter(z_hbm, gidx_hbm, sidx_hbm, out_hbm,
                   gidx_v, sidx_v, rows_a, rows_b, acc, sem_a, sem_b):
        c = lax.axis_index("c")
        s = lax.axis_index("s")
        w = c * NS + s
        zero16 = jnp.zeros((16,), jnp.float32)

        def zrow(i, carry):
            for l in range(outc // 16):
                rows_a[i, pl.ds(l * 16, 16)] = zero16
            return carry

        lax.fori_loop(0, CHUNK, zrow, 0)
        for off in range(0, rows_per_sub, CHUNK):
            sz = min(CHUNK, rows_per_sub - off)
            pltpu.sync_copy(
                rows_a.at[pl.ds(0, sz)],
                acc.at[pl.ds(s * rows_per_sub + off, sz)])
        pltpu.sync_copy(gidx_hbm.at[w], gidx_v)
        pltpu.sync_copy(sidx_hbm.at[w], sidx_v)
        plsc.subcore_barrier()

        pltpu.async_copy(z_hbm.at[gidx_v.at[0]], rows_a, sem_a)

        def body(i, carry):
            j = 2 * i
            pltpu.async_copy(z_hbm.at[gidx_v.at[j + 1]], rows_b, sem_b)
            pltpu.make_async_copy(z_hbm.at[gidx_v.at[j]], rows_a,
                                  sem_a).wait()
            pltpu.sync_copy(rows_a, acc.at[sidx_v.at[j]], add=True)

            @pl.when(j + 2 < n_l)
            def _():
                pltpu.async_copy(z_hbm.at[gidx_v.at[j + 2]], rows_a, sem_a)

            pltpu.make_async_copy(z_hbm.at[gidx_v.at[j + 1]], rows_b,
                                  sem_b).wait()
            pltpu.sync_copy(rows_b, acc.at[sidx_v.at[j + 1]], add=True)
            return carry

        lax.fori_loop(0, n_l // 2, body, 0)
        plsc.subcore_barrier()
        pltpu.sync_copy(acc.at[pl.ds(s * rows_per_sub, rows_per_sub)],
                        out_hbm.at[c, pl.ds(s * rows_per_sub,
                                            rows_per_sub)])

    return sc_scatter


def _make_idx(nbr_in_g, nbr_out_g, n_vox, acc_rows):
    kg, e_tot = nbr_in_g.shape
    n_pairs = kg * e_tot
    pairs_pad = _round_up(n_pairs, NW * CHUNK * 2)
    n_l = pairs_pad // (NW * CHUNK)
    koff = (jnp.arange(kg, dtype=jnp.int32) * n_vox)[:, None]
    gflat = (nbr_in_g.astype(jnp.int32) + koff).reshape(-1)
    pad_src = jnp.arange(pairs_pad - n_pairs, dtype=jnp.int32) % (kg * n_vox)
    gidx = jnp.concatenate([gflat, pad_src]).reshape(NW, n_l, CHUNK)
    n_dump = acc_rows - n_vox
    pad_dst = n_vox + jnp.arange(pairs_pad - n_pairs, dtype=jnp.int32) % n_dump
    sidx = jnp.concatenate(
        [nbr_out_g.astype(jnp.int32).reshape(-1), pad_dst]
    ).reshape(NW, n_l, CHUNK)
    return gidx, sidx, n_l


def kernel(x, nbr_in, nbr_out, kernel):
    n_vox, inc = x.shape
    k_tot, e_tot = nbr_in.shape
    outc = kernel.shape[-1]
    acc_rows = _round_up(n_vox + 1, NS * 8)
    h = k_tot // 3

    za = _matmul_z(x, kernel[:h]).reshape(h * n_vox, outc)
    zb = _matmul_z(x, kernel[h:]).reshape((k_tot - h) * n_vox, outc)

    gidx_a, sidx_a, n_la = _make_idx(nbr_in[:h], nbr_out[:h],
                                     n_vox, acc_rows)
    gidx_b, sidx_b, n_lb = _make_idx(nbr_in[h:], nbr_out[h:],
                                     n_vox, acc_rows)

    pa = _make_sc_scatter(outc, acc_rows, n_la)(za, gidx_a, sidx_a)
    pb = _make_sc_scatter(outc, acc_rows, n_lb)(zb, gidx_b, sidx_b)

    return _add_partials4(pa, pb, n_vox)


# speedup vs baseline: 4.7561x; 1.0124x over previous
import functools

import jax
import jax.numpy as jnp
from jax import lax
from jax.experimental import pallas as pl
from jax.experimental.pallas import tpu as pltpu
from jax.experimental.pallas import tpu_sc as plsc

NC = 2
NS = 16
NW = NC * NS
CHUNK = 128


def _round_up(a, b):
    return (a + b - 1) // b * b


def _matmul_z(x, w):
    v_tot, inc = x.shape
    k_tot, _, outc = w.shape
    vb = 1000 if v_tot % 1000 == 0 else v_tot
    nv = v_tot // vb

    def body(x_ref, w_ref, o_ref):
        xb = x_ref[...].astype(jnp.bfloat16)
        for k in range(k_tot):
            o_ref[k] = jnp.dot(xb, w_ref[k].astype(jnp.bfloat16),
                               preferred_element_type=jnp.float32)

    return pl.pallas_call(
        body,
        grid=(nv,),
        in_specs=[
            pl.BlockSpec((vb, inc), lambda v: (v, 0)),
            pl.BlockSpec((k_tot, inc, outc), lambda v: (0, 0, 0)),
        ],
        out_specs=pl.BlockSpec((k_tot, vb, outc), lambda v: (0, v, 0)),
        out_shape=jax.ShapeDtypeStruct((k_tot, v_tot, outc), jnp.float32),
    )(x, w)


def _add_partials4(pa, pb, n_vox):
    _, _, outc = pa.shape
    vb = 2000 if n_vox % 2000 == 0 else n_vox
    nv = n_vox // vb

    def body(a_ref, b_ref, o_ref):
        o_ref[...] = ((a_ref[0] + a_ref[1])
                      + (b_ref[0] + b_ref[1]))

    spec = pl.BlockSpec((NC, vb, outc), lambda v: (0, v, 0))
    return pl.pallas_call(
        body,
        grid=(nv,),
        in_specs=[spec, spec],
        out_specs=pl.BlockSpec((vb, outc), lambda v: (v, 0)),
        out_shape=jax.ShapeDtypeStruct((n_vox, outc), jnp.float32),
    )(pa, pb)


def _make_sc_scatter(outc, acc_rows, n_l):
    rows_per_sub = acc_rows // NS
    mesh = plsc.VectorSubcoreMesh(core_axis_name="c", subcore_axis_name="s")

    @functools.partial(
        pl.kernel,
        mesh=mesh,
        out_type=jax.ShapeDtypeStruct((NC, acc_rows, outc), jnp.float32),
        scratch_types=[
            pltpu.VMEM((n_l, CHUNK), jnp.int32),
            pltpu.VMEM((n_l, CHUNK), jnp.int32),
            pltpu.VMEM((CHUNK, outc), jnp.float32),
            pltpu.VMEM((CHUNK, outc), jnp.float32),
            pltpu.VMEM_SHARED((acc_rows, outc), jnp.float32),
            pltpu.SemaphoreType.DMA,
            pltpu.SemaphoreType.DMA,
        ],
    )
    def sc_scatter(z_hbm, gidx_hbm, sidx_hbm, out_hbm,
                   gidx_v, sidx_v, rows_a, rows_b, acc, sem_a, sem_b):
        c = lax.axis_index("c")
        s = lax.axis_index("s")
        w = c * NS + s
        zero16 = jnp.zeros((16,), jnp.float32)

        def zrow(i, carry):
            for l in range(outc // 16):
                rows_a[i, pl.ds(l * 16, 16)] = zero16
            return carry

        lax.fori_loop(0, CHUNK, zrow, 0)
        for off in range(0, rows_per_sub, CHUNK):
            sz = min(CHUNK, rows_per_sub - off)
            pltpu.sync_copy(
                rows_a.at[pl.ds(0, sz)],
                acc.at[pl.ds(s * rows_per_sub + off, sz)])
        pltpu.sync_copy(gidx_hbm.at[w], gidx_v)
        pltpu.sync_copy(sidx_hbm.at[w], sidx_v)
        plsc.subcore_barrier()

        pltpu.async_copy(z_hbm.at[gidx_v.at[0]], rows_a, sem_a)

        def body(i, carry):
            j = 2 * i
            pltpu.async_copy(z_hbm.at[gidx_v.at[j + 1]], rows_b, sem_b)
            pltpu.make_async_copy(z_hbm.at[gidx_v.at[j]], rows_a,
                                  sem_a).wait()
            pltpu.sync_copy(rows_a, acc.at[sidx_v.at[j]], add=True)

            @pl.when(j + 2 < n_l)
            def _():
                pltpu.async_copy(z_hbm.at[gidx_v.at[j + 2]], rows_a, sem_a)

            pltpu.make_async_copy(z_hbm.at[gidx_v.at[j + 1]], rows_b,
                                  sem_b).wait()
            pltpu.sync_copy(rows_b, acc.at[sidx_v.at[j + 1]], add=True)
            return carry

        lax.fori_loop(0, n_l // 2, body, 0)
        plsc.subcore_barrier()
        pltpu.sync_copy(acc.at[pl.ds(s * rows_per_sub, rows_per_sub)],
                        out_hbm.at[c, pl.ds(s * rows_per_sub,
                                            rows_per_sub)])

    return sc_scatter


def _make_idx(nbr_in_g, nbr_out_g, n_vox, acc_rows):
    kg, e_tot = nbr_in_g.shape
    n_pairs = kg * e_tot
    pairs_pad = _round_up(n_pairs, NW * CHUNK * 2)
    n_l = pairs_pad // (NW * CHUNK)
    koff = (jnp.arange(kg, dtype=jnp.int32) * n_vox)[:, None]
    gflat = (nbr_in_g.astype(jnp.int32) + koff).reshape(-1)
    pad_src = jnp.arange(pairs_pad - n_pairs, dtype=jnp.int32) % (kg * n_vox)
    gidx = jnp.concatenate([gflat, pad_src]).reshape(NW, n_l, CHUNK)
    n_dump = acc_rows - n_vox
    pad_dst = n_vox + jnp.arange(pairs_pad - n_pairs, dtype=jnp.int32) % n_dump
    sidx = jnp.concatenate(
        [nbr_out_g.astype(jnp.int32).reshape(-1), pad_dst]
    ).reshape(NW, n_l, CHUNK)
    return gidx, sidx, n_l


def kernel(x, nbr_in, nbr_out, kernel):
    n_vox, inc = x.shape
    k_tot, e_tot = nbr_in.shape
    outc = kernel.shape[-1]
    acc_rows = _round_up(n_vox + 1, NS * 8)
    h = k_tot * 2 // 5

    za = _matmul_z(x, kernel[:h]).reshape(h * n_vox, outc)
    zb = _matmul_z(x, kernel[h:]).reshape((k_tot - h) * n_vox, outc)

    gidx_a, sidx_a, n_la = _make_idx(nbr_in[:h], nbr_out[:h],
                                     n_vox, acc_rows)
    gidx_b, sidx_b, n_lb = _make_idx(nbr_in[h:], nbr_out[h:],
                                     n_vox, acc_rows)

    pa = _make_sc_scatter(outc, acc_rows, n_la)(za, gidx_a, sidx_a)
    pb = _make_sc_scatter(outc, acc_rows, n_lb)(zb, gidx_b, sidx_b)

    return _add_partials4(pa, pb, n_vox)
